# Initial kernel scaffold; baseline (speedup 1.0000x reference)
import jax
import jax.numpy as jnp
from jax.experimental import pallas as pl


def kernel(source_nodes, destination_nodes, trans_cascades, edge_times, pub_times, user_state, cas_state, last_update, w_user, b_user, w_cas, b_cas):
    raise NotImplementedError("write your pallas kernel here")


# speedup vs baseline: 21.3824x; 21.3824x over previous
import functools

import jax
import jax.numpy as jnp
from jax import lax
from jax.experimental import pallas as pl
from jax.experimental.pallas import tpu as pltpu
from jax.experimental.pallas import tpu_sc as plsc

N_U = 10000
N_C = 10000
D = 128
TD = 128
E = 160000
NW = 32
EPW = E // NW
NVR = (EPW + 15) // 16
EBUF = NVR * 16
SEGP = 10240
NSEG = 3 * SEGP
SL = NSEG // NW
SLW = SEGP // 10
TAIL = N_U - 9 * SLW
NEGINF = float("-inf")

_mesh = plsc.VectorSubcoreMesh(core_axis_name="c", subcore_axis_name="s")
_sc_params = pltpu.CompilerParams(needs_layout_passes=False,
                                  use_tc_tiling_on_sc=False)


def _wid():
    return lax.axis_index("s") * 2 + lax.axis_index("c")


def _f32(x):
    return jnp.asarray(x, jnp.float32)


def _maxt_partial_body(src, dst, cas, tms, part, src_v, dst_v, cas_v, t_v,
                       tab, sem):
    w = _wid()
    base = w * EPW
    cps = [
        pltpu.async_copy(src.at[pl.ds(base, EPW)], src_v.at[pl.ds(0, EPW)], sem),
        pltpu.async_copy(dst.at[pl.ds(base, EPW)], dst_v.at[pl.ds(0, EPW)], sem),
        pltpu.async_copy(cas.at[pl.ds(base, EPW)], cas_v.at[pl.ds(0, EPW)], sem),
        pltpu.async_copy(tms.at[pl.ds(base, EPW)], t_v.at[pl.ds(0, EPW)], sem),
    ]
    minf = jnp.full((16,), NEGINF, jnp.float32)

    def init_body(j, c):
        tab[pl.ds(j * 16, 16)] = minf
        return c

    lax.fori_loop(0, NSEG // 16, init_body, 0)
    for c in cps:
        c.wait()

    lane = lax.iota(jnp.int32, 16)

    def edge_body(i, c):
        off = i * 16
        inb = (off + lane) < EPW
        t16 = jnp.where(inb, t_v[pl.ds(off, 16)], NEGINF)
        for snum, idv in enumerate((src_v, dst_v, cas_v)):
            ids = jnp.where(inb, idv[pl.ds(off, 16)], SEGP - 1) + snum * SEGP
            ts, iss = plsc.sort_key_val(t16, ids)
            _, lastm = plsc.scan_count(iss)
            cur = plsc.load_gather(tab, [iss])
            plsc.store_scatter(tab, [iss], jnp.maximum(ts, cur), mask=lastm)
        return c

    lax.fori_loop(0, NVR, edge_body, 0)
    pltpu.sync_copy(tab, part.at[w])


def _maxt_merge_body(part, maxt, buf, acc, sem):
    w = _wid()
    base = w * SL
    cps = [pltpu.async_copy(part.at[r, pl.ds(base, SL)], buf.at[r], sem)
           for r in range(NW)]
    for c in cps:
        c.wait()

    def seg_body(j, c):
        o = j * 16
        v = buf[0, pl.ds(o, 16)]
        for r in range(1, NW):
            v = jnp.maximum(v, buf[r, pl.ds(o, 16)])
        acc[pl.ds(o, 16)] = v
        return c

    lax.fori_loop(0, SL // 16, seg_body, 0)
    pltpu.sync_copy(acc, maxt.at[pl.ds(base, SL)])


def _pos_partial_body(src, dst, cas, tms, maxt, partp, src_v, dst_v, cas_v,
                      t_v, maxt_v, ptab, sem):
    w = _wid()
    base = w * EPW
    cps = [
        pltpu.async_copy(src.at[pl.ds(base, EPW)], src_v.at[pl.ds(0, EPW)], sem),
        pltpu.async_copy(dst.at[pl.ds(base, EPW)], dst_v.at[pl.ds(0, EPW)], sem),
        pltpu.async_copy(cas.at[pl.ds(base, EPW)], cas_v.at[pl.ds(0, EPW)], sem),
        pltpu.async_copy(tms.at[pl.ds(base, EPW)], t_v.at[pl.ds(0, EPW)], sem),
        pltpu.async_copy(maxt, maxt_v, sem),
    ]
    mneg = jnp.full((16,), -1, jnp.int32)

    def init_body(j, c):
        ptab[pl.ds(j * 16, 16)] = mneg
        return c

    lax.fori_loop(0, NSEG // 16, init_body, 0)
    for c in cps:
        c.wait()

    lane = lax.iota(jnp.int32, 16)

    def edge_body(i, c):
        off = i * 16
        inb = (off + lane) < EPW
        t16 = jnp.where(inb, t_v[pl.ds(off, 16)], NEGINF)
        pos = jnp.where(inb, base + off + lane, 0)
        for snum, idv in enumerate((src_v, dst_v, cas_v)):
            ids = jnp.where(inb, idv[pl.ds(off, 16)], SEGP - 1) + snum * SEGP
            mt = plsc.load_gather(maxt_v, [ids])
            elig = t16 >= mt
            _, lastm = plsc.scan_count(ids, mask=elig)
            m = lastm & elig
            cur = plsc.load_gather(ptab, [ids])
            plsc.store_scatter(ptab, [ids], jnp.maximum(pos, cur), mask=m)
        return c

    lax.fori_loop(0, NVR, edge_body, 0)
    pltpu.sync_copy(ptab, partp.at[w])


def _winner_body(partp, src, dst, cas, tms, pub, lu, usp, csp,
                 raw0, raw1, raw2, dt_o, t_o, val_o,
                 pbuf, bp_v, bpc_v, es_v, ed_v, ec_v, et_v, ep_v, ls_v, ld_v,
                 r0_v, r1_v, r2_v, dt_v, tv_v, vl_v, g0, g1, g2, sem):
    w = _wid()

    @pl.when(w < 30)
    def _():
        stream = w // 10
        nbase = (w % 10) * SLW
        sbase = stream * SEGP + nbase
        cps = [pltpu.async_copy(partp.at[r, pl.ds(sbase, SLW)], pbuf.at[r], sem)
               for r in range(NW)]
        for c in cps:
            c.wait()

        def merge_body(j, c):
            o = j * 16
            v = pbuf[0, pl.ds(o, 16)]
            for r in range(1, NW):
                v = jnp.maximum(v, pbuf[r, pl.ds(o, 16)])
            bp_v[pl.ds(o, 16)] = v
            bpc_v[pl.ds(o, 16)] = jnp.maximum(v, 0)
            return c

        lax.fori_loop(0, SLW // 16, merge_body, 0)

        CH = 128
        cps = []
        for k in range(SLW // CH):
            s = pl.ds(k * CH, CH)
            idx = bpc_v.at[s]
            cps += [
                pltpu.async_copy(src.at[idx], es_v.at[s], sem),
                pltpu.async_copy(dst.at[idx], ed_v.at[s], sem),
                pltpu.async_copy(cas.at[idx], ec_v.at[s], sem),
                pltpu.async_copy(tms.at[idx], et_v.at[s], sem),
                pltpu.async_copy(pub.at[idx], ep_v.at[s], sem),
            ]
        for c in cps:
            c.wait()
        cps = []
        for k in range(SLW // CH):
            s = pl.ds(k * CH, CH)
            cps += [
                pltpu.async_copy(lu.at[es_v.at[s]], ls_v.at[s], sem),
                pltpu.async_copy(lu.at[ed_v.at[s]], ld_v.at[s], sem),
            ]
        for c in cps:
            c.wait()

        s0m = jnp.broadcast_to(stream == 0, (16,))
        s1m = jnp.broadcast_to(stream == 1, (16,))

        def meta_body(j, c):
            o = pl.ds(j * 16, 16)
            bp16 = bp_v[o]
            valid = bp16 >= 0
            et16 = et_v[o]
            ref_t = jnp.where(s0m, ls_v[o], jnp.where(s1m, ld_v[o], ep_v[o]))
            dt_v[o] = jnp.where(valid, et16 - ref_t, 0.0)
            tv_v[o] = jnp.where(valid, et16, 0.0)
            vl_v[o] = jnp.where(valid, 1.0, 0.0)
            r0_v[o] = jnp.where(valid, es_v[o], N_U)
            r1_v[o] = jnp.where(valid, ed_v[o], N_U)
            r2_v[o] = jnp.where(valid, ec_v[o], N_C)
            return c

        lax.fori_loop(0, SLW // 16, meta_body, 0)
        is_edge = (w % 10) == 9

        @pl.when(jnp.logical_not(is_edge))
        def _():
            pltpu.sync_copy(dt_v, dt_o.at[stream, pl.ds(nbase, SLW)])
            pltpu.sync_copy(tv_v, t_o.at[stream, pl.ds(nbase, SLW)])
            pltpu.sync_copy(vl_v, val_o.at[stream, pl.ds(nbase, SLW)])

        @pl.when(is_edge)
        def _():
            pltpu.sync_copy(dt_v.at[pl.ds(0, TAIL)],
                            dt_o.at[stream, pl.ds(nbase, TAIL)])
            pltpu.sync_copy(tv_v.at[pl.ds(0, TAIL)],
                            t_o.at[stream, pl.ds(nbase, TAIL)])
            pltpu.sync_copy(vl_v.at[pl.ds(0, TAIL)],
                            val_o.at[stream, pl.ds(nbase, TAIL)])

        nrows = jnp.minimum(SLW, N_U - nbase)

        def gbody(b, c):
            o = b * 16
            i0 = r0_v[pl.ds(o, 16)]
            i1 = r1_v[pl.ds(o, 16)]
            i2 = r2_v[pl.ds(o, 16)]
            c0 = pltpu.async_copy(usp.at[i0], g0, sem)
            c1 = pltpu.async_copy(usp.at[i1], g1, sem)
            c2 = pltpu.async_copy(csp.at[i2], g2, sem)
            c0.wait()
            c1.wait()
            c2.wait()
            pltpu.sync_copy(g0, raw0.at[stream, pl.ds(nbase + o, 16)])
            pltpu.sync_copy(g1, raw1.at[stream, pl.ds(nbase + o, 16)])
            pltpu.sync_copy(g2, raw2.at[stream, pl.ds(nbase + o, 16)])
            return c

        lax.fori_loop(0, nrows // 16, gbody, 0)


def _assemble_body(raw0, raw1, raw2, dtb, tb, vb, wref, bref, out):
    s = pl.program_id(0)
    dt = dtb[:, 0]
    v = vb[:, 0]
    te = jnp.cos(dt[:, None] * wref[s][None, :] + bref[s][None, :]) * v[:, None]
    out[0, :, 0:128] = raw0[0]
    out[0, :, 128:256] = raw1[0]
    out[0, :, 256:384] = raw2[0]
    out[0, :, 384:512] = te
    out[0, :, 512:513] = (tb[:, 0] * v)[:, None]


def kernel(source_nodes, destination_nodes, trans_cascades, edge_times,
           pub_times, user_state, cas_state, last_update, w_user, b_user,
           w_cas, b_cas):
    f32 = jnp.float32
    i32 = jnp.int32
    src = source_nodes.astype(i32)
    dst = destination_nodes.astype(i32)
    cas = trans_cascades.astype(i32)

    usp = jnp.concatenate([user_state, jnp.zeros((8, D), f32)], axis=0)
    csp = jnp.concatenate([cas_state, jnp.zeros((8, D), f32)], axis=0)

    k1 = pl.kernel(
        _maxt_partial_body,
        out_type=jax.ShapeDtypeStruct((NW, NSEG), f32),
        mesh=_mesh,
        compiler_params=_sc_params,
        scratch_types=[
            pltpu.VMEM((EBUF,), i32), pltpu.VMEM((EBUF,), i32),
            pltpu.VMEM((EBUF,), i32), pltpu.VMEM((EBUF,), f32),
            pltpu.VMEM((NSEG,), f32), pltpu.SemaphoreType.DMA,
        ],
    )
    part = k1(src, dst, cas, edge_times)

    k2 = pl.kernel(
        _maxt_merge_body,
        out_type=jax.ShapeDtypeStruct((NSEG,), f32),
        mesh=_mesh,
        compiler_params=_sc_params,
        scratch_types=[
            pltpu.VMEM((NW, SL), f32), pltpu.VMEM((SL,), f32),
            pltpu.SemaphoreType.DMA,
        ],
    )
    maxt = k2(part)

    k3 = pl.kernel(
        _pos_partial_body,
        out_type=jax.ShapeDtypeStruct((NW, NSEG), i32),
        mesh=_mesh,
        compiler_params=_sc_params,
        scratch_types=[
            pltpu.VMEM((EBUF,), i32), pltpu.VMEM((EBUF,), i32),
            pltpu.VMEM((EBUF,), i32), pltpu.VMEM((EBUF,), f32),
            pltpu.VMEM((NSEG,), f32), pltpu.VMEM((NSEG,), i32),
            pltpu.SemaphoreType.DMA,
        ],
    )
    partp = k3(src, dst, cas, edge_times, maxt)

    k45 = pl.kernel(
        _winner_body,
        out_type=(
            jax.ShapeDtypeStruct((3, N_U, D), f32),
            jax.ShapeDtypeStruct((3, N_U, D), f32),
            jax.ShapeDtypeStruct((3, N_U, D), f32),
            jax.ShapeDtypeStruct((3, N_U), f32),
            jax.ShapeDtypeStruct((3, N_U), f32),
            jax.ShapeDtypeStruct((3, N_U), f32),
        ),
        mesh=_mesh,
        compiler_params=_sc_params,
        scratch_types=[
            pltpu.VMEM((NW, SLW), i32),
            pltpu.VMEM((SLW,), i32), pltpu.VMEM((SLW,), i32),
            pltpu.VMEM((SLW,), i32), pltpu.VMEM((SLW,), i32),
            pltpu.VMEM((SLW,), i32), pltpu.VMEM((SLW,), f32),
            pltpu.VMEM((SLW,), f32), pltpu.VMEM((SLW,), f32),
            pltpu.VMEM((SLW,), f32),
            pltpu.VMEM((SLW,), i32), pltpu.VMEM((SLW,), i32),
            pltpu.VMEM((SLW,), i32),
            pltpu.VMEM((SLW,), f32), pltpu.VMEM((SLW,), f32),
            pltpu.VMEM((SLW,), f32),
            pltpu.VMEM((16, D), f32), pltpu.VMEM((16, D), f32),
            pltpu.VMEM((16, D), f32),
            pltpu.SemaphoreType.DMA,
        ],
    )
    raw0, raw1, raw2, dt_o, t_o, val_o = k45(
        partp, src, dst, cas, edge_times, pub_times, last_update, usp, csp)

    B = 400
    wu2 = w_user.reshape(1, TD)
    bu2 = b_user.reshape(1, TD)
    wc2 = w_cas.reshape(1, TD)
    bc2 = b_cas.reshape(1, TD)
    wall = jnp.concatenate([wu2, wu2, wc2], axis=0)
    ball = jnp.concatenate([bu2, bu2, bc2], axis=0)

    nb = N_U // B
    dt_c = dt_o.reshape(3 * N_U, 1)
    t_c = t_o.reshape(3 * N_U, 1)
    val_c = val_o.reshape(3 * N_U, 1)

    out = pl.pallas_call(
        _assemble_body,
        out_shape=jax.ShapeDtypeStruct((3, N_U, 513), f32),
        grid=(3, nb),
        in_specs=[
            pl.BlockSpec((1, B, D), lambda s, j: (s, j, 0)),
            pl.BlockSpec((1, B, D), lambda s, j: (s, j, 0)),
            pl.BlockSpec((1, B, D), lambda s, j: (s, j, 0)),
            pl.BlockSpec((B, 1), lambda s, j: (s * nb + j, 0)),
            pl.BlockSpec((B, 1), lambda s, j: (s * nb + j, 0)),
            pl.BlockSpec((B, 1), lambda s, j: (s * nb + j, 0)),
            pl.BlockSpec((3, TD), lambda s, j: (0, 0)),
            pl.BlockSpec((3, TD), lambda s, j: (0, 0)),
        ],
        out_specs=pl.BlockSpec((1, B, 513), lambda s, j: (s, j, 0)),
    )(raw0, raw1, raw2, dt_c, t_c, val_c, wall, ball)
    return out


# speedup vs baseline: 35.8297x; 1.6757x over previous
import jax
import jax.numpy as jnp
from jax import lax
from jax.experimental import pallas as pl
from jax.experimental.pallas import tpu as pltpu
from jax.experimental.pallas import tpu_sc as plsc

N_U = 10000
N_C = 10000
D = 128
TD = 128
E = 160000
NW = 32
EPW = E // NW
NVR = (EPW + 15) // 16
EBUF = NVR * 16
SEGP = 10240
NSEG = 3 * SEGP
SLW = SEGP // 10
TAIL = N_U - 9 * SLW
NTBL = N_U + 8
CTBL = 2 * NTBL
BN = 32
NB_FULL = SLW // BN
NB_EDGE = (TAIL - 16) // BN
NEGINF = float("-inf")

_mesh = plsc.VectorSubcoreMesh(core_axis_name="c", subcore_axis_name="s")
_sc_params = pltpu.CompilerParams(needs_layout_passes=False,
                                  use_tc_tiling_on_sc=False)


def _wid():
    return lax.axis_index("s") * 2 + lax.axis_index("c")


def _scan_body(src, dst, cas, tms, part_t, part_p,
               src_v, dst_v, cas_v, t_v, tt0, tt1, tt2, tp0, tp1, tp2, sem):
    w = _wid()
    base = w * EPW
    cps = [
        pltpu.async_copy(src.at[pl.ds(base, EPW)], src_v.at[pl.ds(0, EPW)], sem),
        pltpu.async_copy(dst.at[pl.ds(base, EPW)], dst_v.at[pl.ds(0, EPW)], sem),
        pltpu.async_copy(cas.at[pl.ds(base, EPW)], cas_v.at[pl.ds(0, EPW)], sem),
        pltpu.async_copy(tms.at[pl.ds(base, EPW)], t_v.at[pl.ds(0, EPW)], sem),
    ]
    minf = jnp.full((16,), NEGINF, jnp.float32)
    mneg = jnp.full((16,), -1, jnp.int32)

    def init_body(j, c):
        s = pl.ds(j * 16, 16)
        tt0[s] = minf
        tt1[s] = minf
        tt2[s] = minf
        tp0[s] = mneg
        tp1[s] = mneg
        tp2[s] = mneg
        return c

    lax.fori_loop(0, SEGP // 16, init_body, 0)
    for c in cps:
        c.wait()

    lane = lax.iota(jnp.int32, 16)

    def edge_body(i, c):
        off = i * 16
        inb = (off + lane) < EPW
        t16 = jnp.where(inb, t_v[pl.ds(off, 16)], NEGINF)
        pos = jnp.where(inb, base + off + lane, 0)
        for idv, tt, tp in ((src_v, tt0, tp0), (dst_v, tt1, tp1),
                            (cas_v, tt2, tp2)):
            ids = jnp.where(inb, idv[pl.ds(off, 16)], SEGP - 1)
            ts, iss = plsc.sort_key_val(t16, ids)
            _, lastm = plsc.scan_count(iss)
            cur = plsc.load_gather(tt, [iss])
            plsc.store_scatter(tt, [iss], jnp.maximum(ts, cur), mask=lastm)
            nm = plsc.load_gather(tt, [ids])
            elig = t16 >= nm
            _, lm2 = plsc.scan_count(ids, mask=elig)
            m2 = lm2 & elig
            curp = plsc.load_gather(tp, [ids])
            plsc.store_scatter(tp, [ids], jnp.maximum(pos, curp), mask=m2)
        return c

    lax.fori_loop(0, NVR, edge_body, 0)
    for s, tt in enumerate((tt0, tt1, tt2)):
        pltpu.sync_copy(tt, part_t.at[w, pl.ds(s * SEGP, SEGP)])
    for s, tp in enumerate((tp0, tp1, tp2)):
        pltpu.sync_copy(tp, part_p.at[w, pl.ds(s * SEGP, SEGP)])


def _winner_body(part_t, part_p, src, dst, cas, tms, pub, lu, tbl,
                 raw0, raw1, raw2, dt_o, t_o, val_o,
                 pt_b, pp_b, bp_v, bpc_v, es_v, ed_v, ec_v, et_v, ep_v,
                 ls_v, ld_v, dt_v, tv_v, vl_v, idx_v, g_a, g_b,
                 sem, gs_a, gs_b, ws_a, ws_b):
    w = _wid()

    @pl.when(w < 30)
    def _():
        stream = w // 10
        nbase = (w % 10) * SLW
        sbase = stream * SEGP + nbase
        is_edge = (w % 10) == 9
        cps = [pltpu.async_copy(part_t.at[r, pl.ds(sbase, SLW)], pt_b.at[r],
                                sem) for r in range(NW)]
        cps += [pltpu.async_copy(part_p.at[r, pl.ds(sbase, SLW)], pp_b.at[r],
                                 sem) for r in range(NW)]
        for c in cps:
            c.wait()

        def merge_body(j, c):
            o = pl.ds(j * 16, 16)
            bt = pt_b[0, o]
            for r in range(1, NW):
                bt = jnp.maximum(bt, pt_b[r, o])
            bp = jnp.full((16,), -1, jnp.int32)
            for r in range(NW):
                bp = jnp.maximum(bp, jnp.where(pt_b[r, o] >= bt, pp_b[r, o],
                                               -1))
            bp_v[o] = bp
            bpc_v[o] = jnp.maximum(bp, 0)
            return c

        lax.fori_loop(0, SLW // 16, merge_body, 0)

        CH = 128
        cps = []
        for k in range(SLW // CH):
            s = pl.ds(k * CH, CH)
            idx = bpc_v.at[s]
            cps += [
                pltpu.async_copy(src.at[idx], es_v.at[s], sem),
                pltpu.async_copy(dst.at[idx], ed_v.at[s], sem),
                pltpu.async_copy(cas.at[idx], ec_v.at[s], sem),
                pltpu.async_copy(tms.at[idx], et_v.at[s], sem),
                pltpu.async_copy(pub.at[idx], ep_v.at[s], sem),
            ]
        for c in cps:
            c.wait()
        cps = []
        for k in range(SLW // CH):
            s = pl.ds(k * CH, CH)
            cps += [
                pltpu.async_copy(lu.at[es_v.at[s]], ls_v.at[s], sem),
                pltpu.async_copy(lu.at[ed_v.at[s]], ld_v.at[s], sem),
            ]
        for c in cps:
            c.wait()

        s0m = jnp.broadcast_to(stream == 0, (16,))
        s1m = jnp.broadcast_to(stream == 1, (16,))

        def meta_body(j, c):
            o = pl.ds(j * 16, 16)
            bp16 = bp_v[o]
            valid = bp16 >= 0
            et16 = et_v[o]
            ref_t = jnp.where(s0m, ls_v[o], jnp.where(s1m, ld_v[o], ep_v[o]))
            dt_v[o] = jnp.where(valid, et16 - ref_t, 0.0)
            tv_v[o] = jnp.where(valid, et16, 0.0)
            vl_v[o] = jnp.where(valid, 1.0, 0.0)
            bi = 96 * (j // 2) + 16 * (j % 2)
            idx_v[pl.ds(bi, 16)] = jnp.where(valid, es_v[o], N_U)
            idx_v[pl.ds(bi + 32, 16)] = jnp.where(valid, ed_v[o], N_U)
            idx_v[pl.ds(bi + 64, 16)] = jnp.where(valid, ec_v[o] + NTBL,
                                                  N_U + NTBL)
            return c

        lax.fori_loop(0, SLW // 16, meta_body, 0)

        @pl.when(jnp.logical_not(is_edge))
        def _():
            pltpu.sync_copy(dt_v, dt_o.at[stream, pl.ds(nbase, SLW)])
            pltpu.sync_copy(tv_v, t_o.at[stream, pl.ds(nbase, SLW)])
            pltpu.sync_copy(vl_v, val_o.at[stream, pl.ds(nbase, SLW)])

        @pl.when(is_edge)
        def _():
            pltpu.sync_copy(dt_v.at[pl.ds(0, TAIL)],
                            dt_o.at[stream, pl.ds(nbase, TAIL)])
            pltpu.sync_copy(tv_v.at[pl.ds(0, TAIL)],
                            t_o.at[stream, pl.ds(nbase, TAIL)])
            pltpu.sync_copy(vl_v.at[pl.ds(0, TAIL)],
                            val_o.at[stream, pl.ds(nbase, TAIL)])

        raws = (raw0, raw1, raw2)

        def fire_gather(b, buf, gs):
            pltpu.async_copy(tbl.at[idx_v.at[pl.ds(b * 96, 96)]], buf, gs)

        def drain_gather(buf, gs):
            pltpu.make_async_copy(tbl.at[pl.ds(0, 96)], buf, gs).wait()

        def fire_writes(b, buf, ws):
            node = nbase + b * BN
            for c in range(3):
                pltpu.async_copy(buf.at[pl.ds(32 * c, 32)],
                                 raws[c].at[stream, pl.ds(node, 32)], ws)

        def drain_writes(buf, ws):
            for c in range(3):
                pltpu.make_async_copy(buf.at[pl.ds(32 * c, 32)],
                                      raws[c].at[stream, pl.ds(nbase, 32)],
                                      ws).wait()

        nb = jnp.where(is_edge, NB_EDGE, NB_FULL)
        fire_gather(0, g_a, gs_a)

        def pair_body(q, c):
            b0 = 2 * q

            @pl.when(q >= 1)
            def _():
                drain_writes(g_b, ws_b)

            fire_gather(b0 + 1, g_b, gs_b)
            drain_gather(g_a, gs_a)
            fire_writes(b0, g_a, ws_a)
            drain_writes(g_a, ws_a)

            @pl.when(b0 + 2 < nb)
            def _():
                fire_gather(b0 + 2, g_a, gs_a)

            drain_gather(g_b, gs_b)
            fire_writes(b0 + 1, g_b, ws_b)
            return c

        lax.fori_loop(0, nb // 2, pair_body, 0)
        drain_writes(g_b, ws_b)

        @pl.when(is_edge)
        def _():
            b = NB_EDGE
            node = nbase + b * BN
            pltpu.async_copy(tbl.at[idx_v.at[pl.ds(b * 96, 96)]], g_a,
                             gs_a).wait()
            for c in range(3):
                pltpu.sync_copy(g_a.at[pl.ds(32 * c, 16)],
                                raws[c].at[stream, pl.ds(node, 16)])


def _te_body(dtb, tb, vb, wref, bref, te_t, tc_t):
    s = pl.program_id(0)
    dt = dtb[s]
    v = vb[s]
    wv = wref[s]
    bv = bref[s]
    te_t[0] = jnp.cos(wv[:, None] * dt[None, :] + bv[:, None]) * v[None, :]
    tc_t[0, 0] = tb[s] * v


def kernel(source_nodes, destination_nodes, trans_cascades, edge_times,
           pub_times, user_state, cas_state, last_update, w_user, b_user,
           w_cas, b_cas):
    f32 = jnp.float32
    i32 = jnp.int32
    src = source_nodes.astype(i32)
    dst = destination_nodes.astype(i32)
    cas = trans_cascades.astype(i32)

    zrow = jnp.zeros((8, D), f32)
    tbl = jnp.concatenate([user_state, zrow, cas_state, zrow], axis=0)

    k1 = pl.kernel(
        _scan_body,
        out_type=(
            jax.ShapeDtypeStruct((NW, NSEG), f32),
            jax.ShapeDtypeStruct((NW, NSEG), i32),
        ),
        mesh=_mesh,
        compiler_params=_sc_params,
        scratch_types=[
            pltpu.VMEM((EBUF,), i32), pltpu.VMEM((EBUF,), i32),
            pltpu.VMEM((EBUF,), i32), pltpu.VMEM((EBUF,), f32),
            pltpu.VMEM((SEGP,), f32), pltpu.VMEM((SEGP,), f32),
            pltpu.VMEM((SEGP,), f32),
            pltpu.VMEM((SEGP,), i32), pltpu.VMEM((SEGP,), i32),
            pltpu.VMEM((SEGP,), i32),
            pltpu.SemaphoreType.DMA,
        ],
    )
    part_t, part_p = k1(src, dst, cas, edge_times)

    k2 = pl.kernel(
        _winner_body,
        out_type=(
            jax.ShapeDtypeStruct((3, N_U, D), f32),
            jax.ShapeDtypeStruct((3, N_U, D), f32),
            jax.ShapeDtypeStruct((3, N_U, D), f32),
            jax.ShapeDtypeStruct((3, N_U), f32),
            jax.ShapeDtypeStruct((3, N_U), f32),
            jax.ShapeDtypeStruct((3, N_U), f32),
        ),
        mesh=_mesh,
        compiler_params=_sc_params,
        scratch_types=[
            pltpu.VMEM((NW, SLW), f32), pltpu.VMEM((NW, SLW), i32),
            pltpu.VMEM((SLW,), i32), pltpu.VMEM((SLW,), i32),
            pltpu.VMEM((SLW,), i32), pltpu.VMEM((SLW,), i32),
            pltpu.VMEM((SLW,), i32), pltpu.VMEM((SLW,), f32),
            pltpu.VMEM((SLW,), f32), pltpu.VMEM((SLW,), f32),
            pltpu.VMEM((SLW,), f32), pltpu.VMEM((SLW,), f32),
            pltpu.VMEM((SLW,), f32), pltpu.VMEM((SLW,), f32),
            pltpu.VMEM((3 * SLW,), i32),
            pltpu.VMEM((96, D), f32), pltpu.VMEM((96, D), f32),
            pltpu.SemaphoreType.DMA, pltpu.SemaphoreType.DMA,
            pltpu.SemaphoreType.DMA, pltpu.SemaphoreType.DMA,
            pltpu.SemaphoreType.DMA,
        ],
    )
    raw0, raw1, raw2, dt_o, t_o, val_o = k2(
        part_t, part_p, src, dst, cas, edge_times, pub_times, last_update,
        tbl)

    wu2 = w_user.reshape(1, TD)
    bu2 = b_user.reshape(1, TD)
    wc2 = w_cas.reshape(1, TD)
    bc2 = b_cas.reshape(1, TD)
    wall = jnp.concatenate([wu2, wu2, wc2], axis=0)
    ball = jnp.concatenate([bu2, bu2, bc2], axis=0)

    te_t, tc_t = pl.pallas_call(
        _te_body,
        out_shape=(
            jax.ShapeDtypeStruct((3, TD, N_U), f32),
            jax.ShapeDtypeStruct((3, 1, N_U), f32),
        ),
        grid=(3,),
        in_specs=[
            pl.BlockSpec((3, N_U), lambda s: (0, 0)),
            pl.BlockSpec((3, N_U), lambda s: (0, 0)),
            pl.BlockSpec((3, N_U), lambda s: (0, 0)),
            pl.BlockSpec((3, TD), lambda s: (0, 0)),
            pl.BlockSpec((3, TD), lambda s: (0, 0)),
        ],
        out_specs=[
            pl.BlockSpec((1, TD, N_U), lambda s: (s, 0, 0)),
            pl.BlockSpec((1, 1, N_U), lambda s: (s, 0, 0)),
        ],
    )(dt_o, t_o, val_o, wall, ball)

    te = jnp.transpose(te_t, (0, 2, 1))
    tcol = jnp.transpose(tc_t, (0, 2, 1))
    return jnp.concatenate([raw0, raw1, raw2, te, tcol], axis=-1)
